# Initial kernel scaffold; baseline (speedup 1.0000x reference)
import jax
import jax.numpy as jnp
from jax.experimental import pallas as pl


def kernel(z, edge_index):
    raise NotImplementedError("write your pallas kernel here")


# speedup vs baseline: 3.5919x; 3.5919x over previous
import functools

import jax
import jax.numpy as jnp
from jax import lax
from jax.experimental import pallas as pl
from jax.experimental.pallas import tpu as pltpu
from jax.experimental.pallas import tpu_sc as plsc

E = 320000
D = 128
NW = 32
E_PER_W = E // NW
B = 400
NCHUNK = E_PER_W // B
NGROUP = B // 16

_mesh = plsc.VectorSubcoreMesh(core_axis_name="c", subcore_axis_name="s")

_SHUFFLE_DNUMS = lax.GatherDimensionNumbers(
    offset_dims=(), collapsed_slice_dims=(0,), start_index_map=(0,))


def _lane_shuffle(x, idx):
    return lax.gather(x, idx[:, None], _SHUFFLE_DNUMS, (1,),
                      mode=lax.GatherScatterMode.PROMISE_IN_BOUNDS)


@functools.partial(
    pl.kernel,
    mesh=_mesh,
    out_type=jax.ShapeDtypeStruct((E,), jnp.float32),
    scratch_types=[
        pltpu.VMEM((B,), jnp.int32),
        pltpu.VMEM((B,), jnp.int32),
        pltpu.VMEM((B, D), jnp.float32),
        pltpu.VMEM((B, D), jnp.float32),
        pltpu.VMEM((B,), jnp.float32),
        pltpu.SemaphoreType.DMA,
        pltpu.SemaphoreType.DMA,
    ],
)
def _decoder_sc(z_hbm, src_hbm, dst_hbm, out_hbm,
                si_v, di_v, zi_v, zj_v, o_v, sem_i, sem_j):
    wid = lax.axis_index("s") * 2 + lax.axis_index("c")
    base = wid * E_PER_W
    lane = lax.iota(jnp.int32, 16)

    def chunk_body(i, carry):
        off = base + i * B
        pltpu.sync_copy(src_hbm.at[pl.ds(off, B)], si_v)
        pltpu.sync_copy(dst_hbm.at[pl.ds(off, B)], di_v)
        cp_i = pltpu.async_copy(z_hbm.at[si_v], zi_v, sem_i)
        cp_j = pltpu.async_copy(z_hbm.at[di_v], zj_v, sem_j)
        cp_i.wait()
        cp_j.wait()

        def group_body(g, c):
            e0 = g * 16
            tot = jnp.zeros((16,), jnp.float32)
            for e16 in range(16):
                e = e0 + e16
                acc = zi_v[e, pl.ds(0, 16)] * zj_v[e, pl.ds(0, 16)]
                for k in range(1, D // 16):
                    acc += (zi_v[e, pl.ds(k * 16, 16)]
                            * zj_v[e, pl.ds(k * 16, 16)])
                for shift in (8, 4, 2, 1):
                    acc = acc + _lane_shuffle(acc, lane ^ shift)
                tot = jnp.where(lane == e16, acc, tot)
            o_v[pl.ds(e0, 16)] = tot
            return c

        lax.fori_loop(0, NGROUP, group_body, 0)
        pltpu.sync_copy(o_v, out_hbm.at[pl.ds(off, B)])
        return carry

    lax.fori_loop(0, NCHUNK, chunk_body, 0)


def kernel(z, edge_index):
    ei = edge_index.astype(jnp.int32)
    return _decoder_sc(z, ei[0], ei[1])
